# baseline (speedup 1.0000x reference)
import functools

import jax
import jax.numpy as jnp
from jax import lax
from jax.experimental import pallas as pl
from jax.experimental.pallas import tpu as pltpu
from jax.experimental.pallas import tpu_sc as plsc

K_TOP = 16
CHUNK = 128
NC = 2
NS = 16
NW = NC * NS


def _relayout_body(xt_ref, lin_ref, m_ref):
    d = xt_ref[0]
    hb, c, w = d.shape
    d = jnp.transpose(d, (0, 2, 1))
    g = hb * w // 4
    d3 = d.reshape(g, 4, c)
    r0, r1, r2, r3 = d3[:, 0], d3[:, 1], d3[:, 2], d3[:, 3]
    o0 = jnp.concatenate([r0, r1[:, 0:32]], axis=1)
    o1 = jnp.concatenate([r1[:, 32:96], r2[:, 0:64]], axis=1)
    o2 = jnp.concatenate([r2[:, 64:96], r3], axis=1)
    out = jnp.stack([o0, o1, o2], axis=1)
    lin = out.reshape(3 * g, CHUNK)
    lin_ref[...] = lin
    m_ref[...] = jnp.max(lin.reshape(3 * g // 128, 128, CHUNK), axis=2)[None]


def _merge_desc_into_asc(s_asc, v_desc):
    return jnp.maximum(s_asc, v_desc)


def _sc_body(m_hbm, x_hbm, out_hbm, m_v, idx_v, rows_v, out_v, sem,
             *, rows_per_w, n_chunks):
    wid = lax.axis_index("s") * NC + lax.axis_index("c")
    iota = lax.iota(jnp.int32, 16)
    n_mvec = (n_chunks + 15) // 16
    tail = n_chunks - (n_mvec - 1) * 16
    neg = jnp.float32(-jnp.inf)

    def row_body(t, carry):
        r = wid * rows_per_w + t
        pltpu.sync_copy(m_hbm.at[pl.ds(r * n_chunks, n_chunks)],
                        m_v.at[pl.ds(0, n_chunks)])

        sk, sv = plsc.sort_key_val(m_v[pl.ds(0, 16)], iota)
        bmin = jnp.min(sk)

        def merge_kv(sk, sv, bmin, v, vidx):
            vk, vv = plsc.sort_key_val(v, vidx, descending=True)
            cond = sk >= vk
            nk = jnp.where(cond, sk, vk)
            nv = jnp.where(cond, sv, vv)
            sk, sv = plsc.sort_key_val(nk, nv)
            return sk, sv, jnp.min(nk)

        def skip_kv(sk, sv, bmin, v, vidx):
            return sk, sv, bmin

        for j in range(1, n_mvec):
            v = m_v[pl.ds(j * 16, 16)]
            if j == n_mvec - 1 and tail < 16:
                v = jnp.where(iota < tail, v, neg)
            sk, sv, bmin = lax.cond(jnp.max(v) > bmin, merge_kv, skip_kv,
                                    sk, sv, bmin, v, iota + (j * 16))

        gidx = lax.rev(sv, (0,)) + r * n_chunks
        idx_v[...] = gidx
        pltpu.async_copy(x_hbm.at[idx_v], rows_v, sem).wait()

        s = lax.sort(rows_v[0, pl.ds(0, 16)])
        smin = jnp.min(s)

        def merge(s, smin, v):
            vd, _ = plsc.sort_key_val(v, iota, descending=True)
            sb = _merge_desc_into_asc(s, vd)
            return lax.sort(sb), jnp.min(sb)

        def skip(s, smin, v):
            return s, smin

        for i in range(K_TOP):
            for j in range(CHUNK // 16):
                if i == 0 and j == 0:
                    continue
                v = rows_v[i, pl.ds(j * 16, 16)]
                mx = jnp.max(v)
                s, smin = lax.cond(mx > smin, merge, skip, s, smin, v)

        mean = jnp.sum(s) * (1.0 / K_TOP)
        out_v[t] = jnp.full((16,), mean, jnp.float32)
        return carry

    lax.fori_loop(0, rows_per_w, row_body, jnp.int32(0))
    pltpu.sync_copy(out_v, out_hbm.at[pl.ds(wid * rows_per_w, rows_per_w)])


def kernel(inputs):
    b, h, w, c = inputs.shape
    hw = h * w
    rows = b * c
    assert hw % CHUNK == 0 and rows % NW == 0
    n_chunks = hw // CHUNK
    m_pad = ((n_chunks + 15) // 16) * 16
    rows_per_w = rows // NW
    n_total = rows * n_chunks

    xt = inputs.transpose(0, 1, 3, 2)
    hb = 16
    n_blk = hb * w * c // CHUNK
    grid_h = h // hb
    n_grid = b * grid_h
    lin, m3 = pl.pallas_call(
        _relayout_body,
        grid=(b, grid_h),
        in_specs=[pl.BlockSpec((1, hb, c, w), lambda i, j: (i, j, 0, 0))],
        out_specs=[
            pl.BlockSpec((n_blk, CHUNK), lambda i, j, gh=grid_h: (i * gh + j, 0)),
            pl.BlockSpec((1, n_blk // 128, 128),
                         lambda i, j, gh=grid_h: (i * gh + j, 0, 0)),
        ],
        out_shape=[
            jax.ShapeDtypeStruct((n_total, CHUNK), jnp.float32),
            jax.ShapeDtypeStruct((n_grid, n_blk // 128, 128), jnp.float32),
        ],
    )(xt)

    mesh = plsc.VectorSubcoreMesh(
        core_axis_name="c", subcore_axis_name="s",
        num_cores=NC, num_subcores=NS)
    sc = pl.kernel(
        functools.partial(_sc_body, rows_per_w=rows_per_w,
                          n_chunks=n_chunks),
        out_type=jax.ShapeDtypeStruct((rows, 16), jnp.float32),
        mesh=mesh,
        scratch_types=[
            pltpu.VMEM((m_pad,), jnp.float32),
            pltpu.VMEM((16,), jnp.int32),
            pltpu.VMEM((K_TOP, CHUNK), jnp.float32),
            pltpu.VMEM((rows_per_w, 16), jnp.float32),
            pltpu.SemaphoreType.DMA,
        ],
        compiler_params=pltpu.CompilerParams(needs_layout_passes=False),
    )
    out16 = sc(m3.reshape(n_total), lin)
    return out16[:, 0].reshape(b, c)


# speedup vs baseline: 1.0190x; 1.0190x over previous
import functools

import jax
import jax.numpy as jnp
from jax import lax
from jax.experimental import pallas as pl
from jax.experimental.pallas import tpu as pltpu
from jax.experimental.pallas import tpu_sc as plsc

K_TOP = 16
CHUNK = 128
NC = 2
NS = 16
NW = NC * NS


def _relayout_body(xt_ref, lin_ref, m_ref):
    d = xt_ref[0]
    hb, c, w = d.shape
    d = jnp.transpose(d, (0, 2, 1))
    g = hb * w // 4
    d3 = d.reshape(g, 4, c)
    r0, r1, r2, r3 = d3[:, 0], d3[:, 1], d3[:, 2], d3[:, 3]
    o0 = jnp.concatenate([r0, r1[:, 0:32]], axis=1)
    o1 = jnp.concatenate([r1[:, 32:96], r2[:, 0:64]], axis=1)
    o2 = jnp.concatenate([r2[:, 64:96], r3], axis=1)
    out = jnp.stack([o0, o1, o2], axis=1)
    lin = out.reshape(3 * g, CHUNK)
    lin_ref[...] = lin
    m_ref[...] = jnp.max(lin.reshape(3 * g // 128, 128, CHUNK), axis=2)[None]


def _merge_desc_into_asc(s_asc, v_desc):
    return jnp.maximum(s_asc, v_desc)


def _sc_body(m_hbm, x_hbm, out_hbm, m_v, idx_v, rows_v, out_v, sem,
             *, rows_per_w, n_chunks):
    wid = lax.axis_index("s") * NC + lax.axis_index("c")
    iota = lax.iota(jnp.int32, 16)
    n_mvec = (n_chunks + 15) // 16
    tail = n_chunks - (n_mvec - 1) * 16
    neg = jnp.float32(-jnp.inf)

    def row_body(t, carry):
        r = wid * rows_per_w + t
        pltpu.sync_copy(m_hbm.at[pl.ds(r * n_chunks, n_chunks)],
                        m_v.at[pl.ds(0, n_chunks)])

        sk, sv = plsc.sort_key_val(m_v[pl.ds(0, 16)], iota)
        for j in range(1, n_mvec):
            v = m_v[pl.ds(j * 16, 16)]
            if j == n_mvec - 1 and tail < 16:
                v = jnp.where(iota < tail, v, neg)
            vk, vv = plsc.sort_key_val(v, iota + (j * 16), descending=True)
            cond = sk >= vk
            nk = jnp.where(cond, sk, vk)
            nv = jnp.where(cond, sv, vv)
            sk, sv = plsc.sort_key_val(nk, nv)

        gidx = lax.rev(sv, (0,)) + r * n_chunks
        idx_v[...] = gidx
        pltpu.async_copy(x_hbm.at[idx_v], rows_v, sem).wait()

        s = lax.sort(rows_v[0, pl.ds(0, 16)])
        smin = jnp.min(s)

        def merge(s, smin, v):
            vd, _ = plsc.sort_key_val(v, iota, descending=True)
            sb = _merge_desc_into_asc(s, vd)
            return lax.sort(sb), jnp.min(sb)

        def skip(s, smin, v):
            return s, smin

        for i in range(K_TOP):
            for j in range(CHUNK // 16):
                if i == 0 and j == 0:
                    continue
                v = rows_v[i, pl.ds(j * 16, 16)]
                mx = jnp.max(v)
                s, smin = lax.cond(mx > smin, merge, skip, s, smin, v)

        mean = jnp.sum(s) * (1.0 / K_TOP)
        out_v[t] = jnp.full((16,), mean, jnp.float32)
        return carry

    lax.fori_loop(0, rows_per_w, row_body, jnp.int32(0))
    pltpu.sync_copy(out_v, out_hbm.at[pl.ds(wid * rows_per_w, rows_per_w)])


def kernel(inputs):
    b, h, w, c = inputs.shape
    hw = h * w
    rows = b * c
    assert hw % CHUNK == 0 and rows % NW == 0
    n_chunks = hw // CHUNK
    m_pad = ((n_chunks + 15) // 16) * 16
    rows_per_w = rows // NW
    n_total = rows * n_chunks

    xt = inputs.transpose(0, 1, 3, 2)
    hb = 16
    n_blk = hb * w * c // CHUNK
    grid_h = h // hb
    n_grid = b * grid_h
    lin, m3 = pl.pallas_call(
        _relayout_body,
        grid=(b, grid_h),
        in_specs=[pl.BlockSpec((1, hb, c, w), lambda i, j: (i, j, 0, 0))],
        out_specs=[
            pl.BlockSpec((n_blk, CHUNK), lambda i, j, gh=grid_h: (i * gh + j, 0)),
            pl.BlockSpec((1, n_blk // 128, 128),
                         lambda i, j, gh=grid_h: (i * gh + j, 0, 0)),
        ],
        out_shape=[
            jax.ShapeDtypeStruct((n_total, CHUNK), jnp.float32),
            jax.ShapeDtypeStruct((n_grid, n_blk // 128, 128), jnp.float32),
        ],
    )(xt)

    mesh = plsc.VectorSubcoreMesh(
        core_axis_name="c", subcore_axis_name="s",
        num_cores=NC, num_subcores=NS)
    sc = pl.kernel(
        functools.partial(_sc_body, rows_per_w=rows_per_w,
                          n_chunks=n_chunks),
        out_type=jax.ShapeDtypeStruct((rows, 16), jnp.float32),
        mesh=mesh,
        scratch_types=[
            pltpu.VMEM((m_pad,), jnp.float32),
            pltpu.VMEM((16,), jnp.int32),
            pltpu.VMEM((K_TOP, CHUNK), jnp.float32),
            pltpu.VMEM((rows_per_w, 16), jnp.float32),
            pltpu.SemaphoreType.DMA,
        ],
        compiler_params=pltpu.CompilerParams(needs_layout_passes=False),
    )
    out16 = sc(m3.reshape(n_total), lin)
    return out16[:, 0].reshape(b, c)
